# baseline (speedup 1.0000x reference)
import functools

import jax
import jax.numpy as jnp
from jax import lax
from jax.experimental import pallas as pl
from jax.experimental.pallas import tpu as pltpu
from jax.experimental.pallas import tpu_sc as plsc

B = 16384
L2_REG_LAMBDA = 0.001

NC = 2
NS = 16
NW = NC * NS
BPW = B // NW
CH = 32
NCHUNK = BPW // CH


def _sc_gather_score(h, t, r, e1, e2, r1, r2, ee, re):
    mesh = plsc.VectorSubcoreMesh(
        core_axis_name="c", subcore_axis_name="s", num_cores=NC, num_subcores=NS
    )

    gather_set = tuple(
        pltpu.VMEM((CH, 128), jnp.float32) for _ in range(9)
    )

    @functools.partial(
        pl.kernel,
        mesh=mesh,
        compiler_params=pltpu.CompilerParams(use_tc_tiling_on_sc=True),
        out_type=(
            jax.ShapeDtypeStruct((B // 8, 128), jnp.float32),
            jax.ShapeDtypeStruct((NW * 8, 128), jnp.float32),
        ),
        scratch_types=(
            pltpu.VMEM((BPW,), jnp.int32),
            pltpu.VMEM((BPW,), jnp.int32),
            pltpu.VMEM((BPW,), jnp.int32),
            pltpu.VMEM((BPW,), jnp.int32),
            pltpu.VMEM((BPW,), jnp.int32),
            pltpu.VMEM((BPW,), jnp.int32),
            pltpu.VMEM((BPW,), jnp.int32),
            pltpu.VMEM((BPW,), jnp.int32),
            pltpu.VMEM((BPW,), jnp.int32),
            gather_set,
            gather_set,
            pltpu.VMEM((BPW // 8, 128), jnp.float32),
            pltpu.VMEM((8, 128), jnp.float32),
            pltpu.SemaphoreType.DMA,
            pltpu.SemaphoreType.DMA,
        ),
    )
    def k(ee_hbm, h_hbm, t_hbm, r_hbm, e1_hbm, e2_hbm, r1_hbm, r2_hbm, re_hbm,
          pacc_hbm, acc_hbm,
          hi, ti, ri, h2, h1, t2, t1, r2i, r1i, bufs_a, bufs_b,
          paccv, accv, sem_a, sem_b):
        wid = lax.axis_index("s") * NC + lax.axis_index("c")
        base = pl.multiple_of(wid * BPW, BPW)
        pltpu.sync_copy(h_hbm.at[pl.ds(base, BPW)], hi)
        pltpu.sync_copy(t_hbm.at[pl.ds(base, BPW)], ti)
        pltpu.sync_copy(r_hbm.at[pl.ds(base, BPW)], ri)

        def widen(m, _):
            sl = pl.ds(m * 16, 16)
            hv = hi[sl]
            tv = ti[sl]
            rv = ri[sl]
            h2[sl] = lax.shift_right_logical(hv, 2)
            h1[sl] = lax.shift_right_logical(hv, 1)
            t2[sl] = lax.shift_right_logical(tv, 2)
            t1[sl] = lax.shift_right_logical(tv, 1)
            r2i[sl] = lax.shift_right_logical(rv, 2)
            r1i[sl] = lax.shift_right_logical(rv, 1)
            return 0

        lax.fori_loop(0, BPW // 16, widen, 0)

        bufs = (bufs_a, bufs_b)
        sems = (sem_a, sem_b)

        def fire(c, b):
            e1h, e2h, e1t, e2t, r1g, r2g, ehg, etg, erg = bufs[b]
            sem = sems[b]
            sl = pl.ds(c * CH, CH)
            pltpu.async_copy(ee_hbm.at[h1.at[sl]], ehg, sem)
            pltpu.async_copy(ee_hbm.at[t1.at[sl]], etg, sem)
            pltpu.async_copy(e1_hbm.at[h2.at[sl]], e1h, sem)
            pltpu.async_copy(e2_hbm.at[h2.at[sl]], e2h, sem)
            pltpu.async_copy(e1_hbm.at[t2.at[sl]], e1t, sem)
            pltpu.async_copy(e2_hbm.at[t2.at[sl]], e2t, sem)
            pltpu.async_copy(r1_hbm.at[r2i.at[sl]], r1g, sem)
            pltpu.async_copy(r2_hbm.at[r2i.at[sl]], r2g, sem)
            pltpu.async_copy(re_hbm.at[r1i.at[sl]], erg, sem)

        def wait_set(b):
            e1h, e2h, e1t, e2t, r1g, r2g, ehg, etg, erg = bufs[b]
            sem = sems[b]
            for dst in (ehg, etg, e1h, e2h, e1t, e2t, r1g, r2g, erg):
                pltpu.make_async_copy(ee_hbm.at[pl.ds(0, CH)], dst, sem).wait()

        def compute(c, b, a32, a64):
            e1h, e2h, e1t, e2t, r1g, r2g, ehg, etg, erg = bufs[b]

            def group(g, carry):
                a32, a64 = carry
                g16 = pl.multiple_of(g * 16, 16)
                hv = hi[pl.ds(c * CH + g16, 16)]
                tv = ti[pl.ds(c * CH + g16, 16)]
                rv = ri[pl.ds(c * CH + g16, 16)]
                for j in range(16):
                    i = g16 + j
                    hj = hv[j]
                    tj = tv[j]
                    rj = rv[j]
                    oh32 = lax.shift_left((hj & 3), 5)
                    ot32 = lax.shift_left((tj & 3), 5)
                    or32 = lax.shift_left((rj & 3), 5)
                    oh64 = lax.shift_left((hj & 1), 6)
                    ot64 = lax.shift_left((tj & 1), 6)
                    or64 = lax.shift_left((rj & 1), 6)
                    s = None
                    for half in range(2):
                        o = half * 16
                        a1 = e1h[i, pl.ds(oh32 + o, 16)]
                        a2 = e2h[i, pl.ds(oh32 + o, 16)]
                        b1 = e1t[i, pl.ds(ot32 + o, 16)]
                        b2 = e2t[i, pl.ds(ot32 + o, 16)]
                        q1 = r1g[i, pl.ds(or32 + o, 16)]
                        q2 = r2g[i, pl.ds(or32 + o, 16)]
                        contrib = ((a1 * b1 + a2 * b2) * q1
                                   + (a1 * b2 - a2 * b1) * q2)
                        s = contrib if s is None else s + contrib
                        sq = (((a1 * a1 + a2 * a2) + (b1 * b1 + b2 * b2))
                              + (q1 * q1 + q2 * q2))
                        a32 = a32 + sq
                    d64 = None
                    for q in range(4):
                        o = q * 16
                        hh = ehg[i, pl.ds(oh64 + o, 16)]
                        tt = etg[i, pl.ds(ot64 + o, 16)]
                        rr = erg[i, pl.ds(or64 + o, 16)]
                        s = s + hh * tt * rr
                        sq = (hh * hh + tt * tt) + rr * rr
                        d64 = sq if d64 is None else d64 + sq
                    a64 = a64 + d64
                    row = c * (CH // 8) + (g16 + j) // 8
                    paccv[row, pl.ds((j & 7) * 16, 16)] = s
                return a32, a64

            return lax.fori_loop(0, CH // 16, group, (a32, a64))

        fire(0, 0)

        def ring(cc, carry):
            a32, a64 = carry
            c0 = cc * 2
            c1 = cc * 2 + 1
            c2 = lax.rem(cc * 2 + 2, NCHUNK)
            fire(c1, 1)
            wait_set(0)
            a32, a64 = compute(c0, 0, a32, a64)
            fire(c2, 0)
            wait_set(1)
            a32, a64 = compute(c1, 1, a32, a64)
            return a32, a64

        acc32, acc64 = lax.fori_loop(
            0, NCHUNK // 2, ring,
            (jnp.zeros((16,), jnp.float32), jnp.zeros((16,), jnp.float32)))
        wait_set(0)

        zero = jnp.zeros((16,), jnp.float32)
        for jrow in range(8):
            accv[jrow, pl.ds(0, 16)] = acc32
            accv[jrow, pl.ds(16, 16)] = acc64
            for lz in range(2, 8):
                accv[jrow, pl.ds(lz * 16, 16)] = zero
        pltpu.sync_copy(paccv, pacc_hbm.at[pl.ds(wid * (BPW // 8), BPW // 8)])
        pltpu.sync_copy(accv, acc_hbm.at[pl.ds(wid * 8, 8)])

    return k(ee.reshape(50000, 128), h, t, r,
             e1.reshape(25000, 128), e2.reshape(25000, 128),
             r1.reshape(250, 128), r2.reshape(250, 128),
             re.reshape(500, 128))


def _tc_loss(pacc, y, accs):
    def body(p_ref, y_ref, a_ref, o_ref):
        x = p_ref[...]
        rows = lax.broadcasted_iota(jnp.int32, (128, 8), 0)
        cols = lax.broadcasted_iota(jnp.int32, (128, 8), 1)
        m = jnp.where(rows // 16 == cols, 1.0, 0.0).astype(jnp.float32)
        s = jax.lax.dot_general(
            x, m, (((1,), (0,)), ((), ())),
            preferred_element_type=jnp.float32)
        z = y_ref[...] * (-s)
        sp = jnp.maximum(z, 0.0) + jnp.log1p(jnp.exp(-jnp.abs(z)))
        loss_f = jnp.sum(sp) * (1.0 / B)
        s32 = jnp.sum(a_ref[:, 0:16]) * 0.125
        s64 = jnp.sum(a_ref[:, 16:32]) * 0.125
        o_ref[0, 0] = loss_f + L2_REG_LAMBDA * (
            s32 * (1.0 / (B * 32.0)) + s64 * (1.0 / (B * 64.0))
        )

    out = pl.pallas_call(
        body,
        out_shape=jax.ShapeDtypeStruct((1, 1), jnp.float32),
        out_specs=pl.BlockSpec(memory_space=pltpu.SMEM),
    )(pacc, y.reshape(B // 8, 8), accs)
    return out.reshape(())


def kernel(h, t, r, input_y, ent_embeddings_1, ent_embeddings_2,
           rel_embeddings_1, rel_embeddings_2, ent_embeddings, rel_embeddings):
    pacc, accs = _sc_gather_score(
        h, t, r, ent_embeddings_1, ent_embeddings_2,
        rel_embeddings_1, rel_embeddings_2, ent_embeddings, rel_embeddings)
    return _tc_loss(pacc, input_y, accs)


# speedup vs baseline: 1.1124x; 1.1124x over previous
import functools

import jax
import jax.numpy as jnp
from jax import lax
from jax.experimental import pallas as pl
from jax.experimental.pallas import tpu as pltpu
from jax.experimental.pallas import tpu_sc as plsc

B = 16384
L2_REG_LAMBDA = 0.001

NC = 2
NS = 16
NW = NC * NS
BPW = B // NW
CH = 128
NCHUNK = BPW // CH


def _sc_gather_score(h, t, r, e1, e2, r1, r2, ee, re):
    mesh = plsc.VectorSubcoreMesh(
        core_axis_name="c", subcore_axis_name="s", num_cores=NC, num_subcores=NS
    )

    gather_set = (
        pltpu.VMEM((CH, 32), jnp.float32),
        pltpu.VMEM((CH, 32), jnp.float32),
        pltpu.VMEM((CH, 32), jnp.float32),
        pltpu.VMEM((CH, 32), jnp.float32),
        pltpu.VMEM((CH, 32), jnp.float32),
        pltpu.VMEM((CH, 32), jnp.float32),
        pltpu.VMEM((CH, 64), jnp.float32),
        pltpu.VMEM((CH, 64), jnp.float32),
        pltpu.VMEM((CH, 64), jnp.float32),
    )

    @functools.partial(
        pl.kernel,
        mesh=mesh,
        compiler_params=pltpu.CompilerParams(use_tc_tiling_on_sc=False),
        out_type=(
            jax.ShapeDtypeStruct((B, 16), jnp.float32),
            jax.ShapeDtypeStruct((2, NW * 16), jnp.float32),
        ),
        scratch_types=(
            pltpu.VMEM((BPW,), jnp.int32),
            pltpu.VMEM((BPW,), jnp.int32),
            pltpu.VMEM((BPW,), jnp.int32),
            gather_set,
            gather_set,
            pltpu.VMEM((CH, 16), jnp.float32),
            pltpu.VMEM((CH, 16), jnp.float32),
            pltpu.VMEM((2, 16), jnp.float32),
            pltpu.SemaphoreType.DMA,
            pltpu.SemaphoreType.DMA,
            pltpu.SemaphoreType.DMA,
        ),
    )
    def k(ee_hbm, h_hbm, t_hbm, r_hbm, e1_hbm, e2_hbm, r1_hbm, r2_hbm, re_hbm,
          pacc_hbm, acc_hbm,
          hi, ti, ri, bufs_a, bufs_b, pacc_a, pacc_b, accv,
          sem_a, sem_b, sem_o):
        wid = lax.axis_index("s") * NC + lax.axis_index("c")
        base = pl.multiple_of(wid * BPW, BPW)
        pltpu.sync_copy(h_hbm.at[pl.ds(base, BPW)], hi)
        pltpu.sync_copy(t_hbm.at[pl.ds(base, BPW)], ti)
        pltpu.sync_copy(r_hbm.at[pl.ds(base, BPW)], ri)

        bufs = (bufs_a, bufs_b)
        paccs = (pacc_a, pacc_b)
        sems = (sem_a, sem_b)

        def fire(c):
            e1h, e2h, e1t, e2t, r1g, r2g, ehg, etg, erg = bufs[c % 2]
            sem = sems[c % 2]
            hs = hi.at[pl.ds(c * CH, CH)]
            ts = ti.at[pl.ds(c * CH, CH)]
            rs = ri.at[pl.ds(c * CH, CH)]
            return (
                pltpu.async_copy(ee_hbm.at[hs], ehg, sem),
                pltpu.async_copy(ee_hbm.at[ts], etg, sem),
                pltpu.async_copy(e1_hbm.at[hs], e1h, sem),
                pltpu.async_copy(e2_hbm.at[hs], e2h, sem),
                pltpu.async_copy(e1_hbm.at[ts], e1t, sem),
                pltpu.async_copy(e2_hbm.at[ts], e2t, sem),
                pltpu.async_copy(r1_hbm.at[rs], r1g, sem),
                pltpu.async_copy(r2_hbm.at[rs], r2g, sem),
                pltpu.async_copy(re_hbm.at[rs], erg, sem),
            )

        acc32 = jnp.zeros((16,), jnp.float32)
        acc64 = jnp.zeros((16,), jnp.float32)
        cps = fire(0)
        out_cp = None
        for c in range(NCHUNK):
            nxt = fire(c + 1) if c + 1 < NCHUNK else ()
            for cp in cps:
                cp.wait()
            cps = nxt
            e1h, e2h, e1t, e2t, r1g, r2g, ehg, etg, erg = bufs[c % 2]
            pacc = paccs[c % 2]

            def row(i, carry, e1h=e1h, e2h=e2h, e1t=e1t, e2t=e2t,
                    r1g=r1g, r2g=r2g, ehg=ehg, etg=etg, erg=erg, pacc=pacc):
                a32, a64 = carry
                s = None
                for half in range(2):
                    sl = pl.ds(half * 16, 16)
                    a1 = e1h[i, sl]
                    a2 = e2h[i, sl]
                    b1 = e1t[i, sl]
                    b2 = e2t[i, sl]
                    q1 = r1g[i, sl]
                    q2 = r2g[i, sl]
                    contrib = ((a1 * b1 + a2 * b2) * q1
                               + (a1 * b2 - a2 * b1) * q2)
                    s = contrib if s is None else s + contrib
                    sq = (((a1 * a1 + a2 * a2) + (b1 * b1 + b2 * b2))
                          + (q1 * q1 + q2 * q2))
                    a32 = a32 + sq
                d64 = None
                for q in range(4):
                    sl = pl.ds(q * 16, 16)
                    hh = ehg[i, sl]
                    tt = etg[i, sl]
                    rr = erg[i, sl]
                    s = s + hh * tt * rr
                    sq = (hh * hh + tt * tt) + rr * rr
                    d64 = sq if d64 is None else d64 + sq
                a64 = a64 + d64
                pacc[i, :] = s
                return a32, a64

            acc32, acc64 = lax.fori_loop(0, CH, row, (acc32, acc64))
            if out_cp is not None:
                out_cp.wait()
            out_cp = pltpu.async_copy(
                pacc, pacc_hbm.at[pl.ds(base + c * CH, CH)], sem_o)
        out_cp.wait()
        accv[0, :] = acc32
        accv[1, :] = acc64
        lane = pl.multiple_of(wid * 16, 16)
        pltpu.sync_copy(accv.at[0], acc_hbm.at[0, pl.ds(lane, 16)])
        pltpu.sync_copy(accv.at[1], acc_hbm.at[1, pl.ds(lane, 16)])

    return k(ee, h, t, r, e1, e2, r1, r2, re)


def _tc_loss(pacc, y, accs):
    def body(p_ref, y_ref, a_ref, o_ref):
        x = p_ref[...]
        rows = lax.broadcasted_iota(jnp.int32, (2048, 128), 0)
        cols = lax.broadcasted_iota(jnp.int32, (2048, 128), 1)
        m = jnp.where(rows // 16 == cols, 1.0, 0.0).astype(jnp.float32)
        s = jax.lax.dot_general(
            x, m, (((1,), (0,)), ((), ())),
            preferred_element_type=jnp.float32)
        z = y_ref[...] * (-s)
        sp = jnp.maximum(z, 0.0) + jnp.log1p(jnp.exp(-jnp.abs(z)))
        loss_f = jnp.sum(sp) * (1.0 / B)
        s32 = jnp.sum(a_ref[0:1, :])
        s64 = jnp.sum(a_ref[1:2, :])
        o_ref[0, 0] = loss_f + L2_REG_LAMBDA * (
            s32 * (1.0 / (B * 32.0)) + s64 * (1.0 / (B * 64.0))
        )

    out = pl.pallas_call(
        body,
        out_shape=jax.ShapeDtypeStruct((1, 1), jnp.float32),
        out_specs=pl.BlockSpec(memory_space=pltpu.SMEM),
    )(pacc.reshape(128, 2048), y.reshape(128, 128), accs)
    return out.reshape(())


def kernel(h, t, r, input_y, ent_embeddings_1, ent_embeddings_2,
           rel_embeddings_1, rel_embeddings_2, ent_embeddings, rel_embeddings):
    pacc, accs = _sc_gather_score(
        h, t, r, ent_embeddings_1, ent_embeddings_2,
        rel_embeddings_1, rel_embeddings_2, ent_embeddings, rel_embeddings)
    return _tc_loss(pacc, input_y, accs)


# speedup vs baseline: 1.2549x; 1.1281x over previous
import functools

import jax
import jax.numpy as jnp
from jax import lax
from jax.experimental import pallas as pl
from jax.experimental.pallas import tpu as pltpu
from jax.experimental.pallas import tpu_sc as plsc

B = 16384
L2_REG_LAMBDA = 0.001

NC = 2
NS = 16
NW = NC * NS
BPW = B // NW
CH = 128
NCHUNK = BPW // CH


def _sc_gather_score(h, t, r, etab, rtab):
    mesh = plsc.VectorSubcoreMesh(
        core_axis_name="c", subcore_axis_name="s", num_cores=NC, num_subcores=NS
    )

    gather_set = (
        pltpu.VMEM((CH, 128), jnp.float32),
        pltpu.VMEM((CH, 128), jnp.float32),
        pltpu.VMEM((CH, 128), jnp.float32),
    )

    @functools.partial(
        pl.kernel,
        mesh=mesh,
        compiler_params=pltpu.CompilerParams(use_tc_tiling_on_sc=False),
        out_type=(
            jax.ShapeDtypeStruct((B, 16), jnp.float32),
            jax.ShapeDtypeStruct((2, NW * 16), jnp.float32),
        ),
        scratch_types=(
            pltpu.VMEM((BPW,), jnp.int32),
            pltpu.VMEM((BPW,), jnp.int32),
            pltpu.VMEM((BPW,), jnp.int32),
            gather_set,
            gather_set,
            pltpu.VMEM((CH, 16), jnp.float32),
            pltpu.VMEM((CH, 16), jnp.float32),
            pltpu.VMEM((2, 16), jnp.float32),
            pltpu.SemaphoreType.DMA,
            pltpu.SemaphoreType.DMA,
            pltpu.SemaphoreType.DMA,
        ),
    )
    def k(e_hbm, r_hbm_tab, h_hbm, t_hbm, r_hbm,
          pacc_hbm, acc_hbm,
          hi, ti, ri, bufs_a, bufs_b, pacc_a, pacc_b, accv,
          sem_a, sem_b, sem_o):
        wid = lax.axis_index("s") * NC + lax.axis_index("c")
        base = pl.multiple_of(wid * BPW, BPW)
        pltpu.sync_copy(h_hbm.at[pl.ds(base, BPW)], hi)
        pltpu.sync_copy(t_hbm.at[pl.ds(base, BPW)], ti)
        pltpu.sync_copy(r_hbm.at[pl.ds(base, BPW)], ri)

        bufs = (bufs_a, bufs_b)
        paccs = (pacc_a, pacc_b)
        sems = (sem_a, sem_b)

        def fire(c):
            eh, et, rr = bufs[c % 2]
            sem = sems[c % 2]
            hs = hi.at[pl.ds(c * CH, CH)]
            ts = ti.at[pl.ds(c * CH, CH)]
            rs = ri.at[pl.ds(c * CH, CH)]
            return (
                pltpu.async_copy(e_hbm.at[hs], eh, sem),
                pltpu.async_copy(e_hbm.at[ts], et, sem),
                pltpu.async_copy(r_hbm_tab.at[rs], rr, sem),
            )

        acc32 = jnp.zeros((16,), jnp.float32)
        acc64 = jnp.zeros((16,), jnp.float32)
        cps = fire(0)
        out_cp = None
        for c in range(NCHUNK):
            nxt = fire(c + 1) if c + 1 < NCHUNK else ()
            for cp in cps:
                cp.wait()
            cps = nxt
            eh, et, rr = bufs[c % 2]
            pacc = paccs[c % 2]

            def row(i, carry, eh=eh, et=et, rr=rr, pacc=pacc):
                a32, a64 = carry
                s = None
                for half in range(2):
                    o = half * 16
                    a1 = eh[i, pl.ds(o, 16)]
                    a2 = eh[i, pl.ds(32 + o, 16)]
                    b1 = et[i, pl.ds(o, 16)]
                    b2 = et[i, pl.ds(32 + o, 16)]
                    q1 = rr[i, pl.ds(o, 16)]
                    q2 = rr[i, pl.ds(32 + o, 16)]
                    contrib = ((a1 * b1 + a2 * b2) * q1
                               + (a1 * b2 - a2 * b1) * q2)
                    s = contrib if s is None else s + contrib
                    sq = (((a1 * a1 + a2 * a2) + (b1 * b1 + b2 * b2))
                          + (q1 * q1 + q2 * q2))
                    a32 = a32 + sq
                d64 = None
                for q in range(4):
                    sl = pl.ds(64 + q * 16, 16)
                    hh = eh[i, sl]
                    tt = et[i, sl]
                    rv = rr[i, sl]
                    s = s + hh * tt * rv
                    sq = (hh * hh + tt * tt) + rv * rv
                    d64 = sq if d64 is None else d64 + sq
                a64 = a64 + d64
                pacc[i, :] = s
                return a32, a64

            acc32, acc64 = lax.fori_loop(0, CH, row, (acc32, acc64))
            if out_cp is not None:
                out_cp.wait()
            out_cp = pltpu.async_copy(
                pacc, pacc_hbm.at[pl.ds(base + c * CH, CH)], sem_o)
        out_cp.wait()
        accv[0, :] = acc32
        accv[1, :] = acc64
        lane = pl.multiple_of(wid * 16, 16)
        pltpu.sync_copy(accv.at[0], acc_hbm.at[0, pl.ds(lane, 16)])
        pltpu.sync_copy(accv.at[1], acc_hbm.at[1, pl.ds(lane, 16)])

    return k(etab, rtab, h, t, r)


def _tc_loss(pacc, y, accs):
    def body(p_ref, y_ref, a_ref, o_ref):
        x = p_ref[...]
        rows = lax.broadcasted_iota(jnp.int32, (2048, 128), 0)
        cols = lax.broadcasted_iota(jnp.int32, (2048, 128), 1)
        m = jnp.where(rows // 16 == cols, 1.0, 0.0).astype(jnp.float32)
        s = jax.lax.dot_general(
            x, m, (((1,), (0,)), ((), ())),
            preferred_element_type=jnp.float32)
        z = y_ref[...] * (-s)
        sp = jnp.maximum(z, 0.0) + jnp.log1p(jnp.exp(-jnp.abs(z)))
        loss_f = jnp.sum(sp) * (1.0 / B)
        s32 = jnp.sum(a_ref[0:1, :])
        s64 = jnp.sum(a_ref[1:2, :])
        o_ref[0, 0] = loss_f + L2_REG_LAMBDA * (
            s32 * (1.0 / (B * 32.0)) + s64 * (1.0 / (B * 64.0))
        )

    out = pl.pallas_call(
        body,
        out_shape=jax.ShapeDtypeStruct((1, 1), jnp.float32),
        out_specs=pl.BlockSpec(memory_space=pltpu.SMEM),
    )(pacc.reshape(128, 2048), y.reshape(128, 128), accs)
    return out.reshape(())


def kernel(h, t, r, input_y, ent_embeddings_1, ent_embeddings_2,
           rel_embeddings_1, rel_embeddings_2, ent_embeddings, rel_embeddings):
    etab = jnp.concatenate(
        [ent_embeddings_1, ent_embeddings_2, ent_embeddings], axis=1)
    rtab = jnp.concatenate(
        [rel_embeddings_1, rel_embeddings_2, rel_embeddings], axis=1)
    pacc, accs = _sc_gather_score(h, t, r, etab, rtab)
    return _tc_loss(pacc, input_y, accs)
